# baseline (speedup 1.0000x reference)
import functools

import jax
import jax.numpy as jnp
from jax import lax
from jax.experimental import pallas as pl
from jax.experimental.pallas import tpu as pltpu

_ALPHA = 0.3


def _esn_fused_kernel(x_ref, h0_ref, win_ref, wr_ref, wout_ref,
                      h_seq_ref, out_ref, h_carry, *, tt):
    c = pl.program_id(0)

    b, r = h0_ref.shape
    n_in = x_ref.shape[2]

    @pl.when(c == 0)
    def _():
        h_carry[...] = h0_ref[...]

    wr = wr_ref[...]
    win_bf = win_ref[...].astype(jnp.bfloat16)
    wout_bf = wout_ref[...].astype(jnp.bfloat16)
    om_a = jnp.float32(1.0 - _ALPHA)
    a = jnp.float32(_ALPHA)

    def body(s, h):
        pre_in = jnp.dot(x_ref[s].astype(jnp.bfloat16), win_bf,
                         preferred_element_type=jnp.float32)
        pre = pre_in + jnp.dot(h, wr, preferred_element_type=jnp.float32)
        h_new = h * om_a + a * jnp.tanh(pre)
        h_seq_ref[s] = h_new
        return h_new

    th = tt // 2
    h_mid = lax.fori_loop(0, th, body, h_carry[...], unroll=True)

    out_ref[:th] = jnp.dot(
        h_seq_ref[:th].reshape(th * b, r).astype(jnp.bfloat16),
        wout_bf,
        preferred_element_type=jnp.float32).reshape(th, b, n_in)

    h_final = lax.fori_loop(th, tt, body, h_mid, unroll=True)
    h_carry[...] = h_final

    out_ref[th:] = jnp.dot(
        h_seq_ref[th:].reshape(th * b, r).astype(jnp.bfloat16),
        wout_bf,
        preferred_element_type=jnp.float32).reshape(th, b, n_in)


@jax.jit
def _esn_forward(x_seq, h0, win_t, wr_t, wout_t):
    T, B, n_in = x_seq.shape
    R = h0.shape[-1]
    tt = 16
    nc = T // tt

    h_seq, out_seq = pl.pallas_call(
        functools.partial(_esn_fused_kernel, tt=tt),
        out_shape=[
            jax.ShapeDtypeStruct((T, B, R), jnp.float32),
            jax.ShapeDtypeStruct((T, B, n_in), jnp.float32),
        ],
        grid=(nc,),
        in_specs=[
            pl.BlockSpec((tt, B, n_in), lambda c: (c, 0, 0)),
            pl.BlockSpec((B, R), lambda c: (0, 0)),
            pl.BlockSpec((n_in, R), lambda c: (0, 0)),
            pl.BlockSpec((R, R), lambda c: (0, 0)),
            pl.BlockSpec((R, n_in), lambda c: (0, 0)),
        ],
        out_specs=[
            pl.BlockSpec((tt, B, R), lambda c: (c, 0, 0)),
            pl.BlockSpec((tt, B, n_in), lambda c: (c, 0, 0)),
        ],
        scratch_shapes=[
            pltpu.VMEM((B, R), jnp.float32),
        ],
        compiler_params=pltpu.CompilerParams(
            dimension_semantics=("arbitrary",)),
    )(x_seq, h0, win_t, wr_t, wout_t)
    return out_seq, h_seq


def kernel(x_seq, h0, win_t, wr_t, wout_t):
    return _esn_forward(x_seq, h0, win_t, wr_t, wout_t)


# speedup vs baseline: 1.1781x; 1.1781x over previous
import functools

import jax
import jax.numpy as jnp
from jax import lax
from jax.experimental import pallas as pl
from jax.experimental.pallas import tpu as pltpu

_ALPHA = 0.3


def _esn_fused_kernel(x_ref, h0_ref, win_ref, wr_ref, wout_ref,
                      h_seq_ref, out_ref, h_carry, w_cat_ref, wout_pad,
                      *, tt):
    c = pl.program_id(0)

    b, r = h0_ref.shape
    n_in = x_ref.shape[2]

    @pl.when(c == 0)
    def _():
        h_carry[...] = h0_ref[...]
        w_cat_ref[:r, :] = wr_ref[...]
        w_cat_ref[r:, :] = win_ref[...].astype(jnp.bfloat16).astype(
            jnp.float32)
        wout_pad[:, :n_in] = wout_ref[...].astype(jnp.bfloat16)
        wout_pad[:, n_in:] = jnp.zeros((r, n_in), jnp.bfloat16)

    w_cat = w_cat_ref[...]
    wout_bf = wout_pad[...]
    om_a = jnp.float32(1.0 - _ALPHA)
    a = jnp.float32(_ALPHA)

    def body(s, h):
        x_s = x_ref[s].astype(jnp.bfloat16).astype(jnp.float32)
        pre = jnp.dot(jnp.concatenate([h, x_s], axis=1), w_cat,
                      preferred_element_type=jnp.float32)
        h_new = h * om_a + a * jnp.tanh(pre)
        h_seq_ref[s] = h_new
        return h_new

    th = tt // 2
    h_mid = lax.fori_loop(0, th, body, h_carry[...], unroll=True)

    out_ref[:th] = jnp.dot(
        h_seq_ref[:th].reshape(th * b, r).astype(jnp.bfloat16),
        wout_bf,
        preferred_element_type=jnp.float32)[:, :n_in].reshape(th, b, n_in)

    h_final = lax.fori_loop(th, tt, body, h_mid, unroll=True)
    h_carry[...] = h_final

    out_ref[th:] = jnp.dot(
        h_seq_ref[th:].reshape(th * b, r).astype(jnp.bfloat16),
        wout_bf,
        preferred_element_type=jnp.float32)[:, :n_in].reshape(th, b, n_in)


@jax.jit
def _esn_forward(x_seq, h0, win_t, wr_t, wout_t):
    T, B, n_in = x_seq.shape
    R = h0.shape[-1]
    tt = 16
    nc = T // tt

    h_seq, out_seq = pl.pallas_call(
        functools.partial(_esn_fused_kernel, tt=tt),
        out_shape=[
            jax.ShapeDtypeStruct((T, B, R), jnp.float32),
            jax.ShapeDtypeStruct((T, B, n_in), jnp.float32),
        ],
        grid=(nc,),
        in_specs=[
            pl.BlockSpec((tt, B, n_in), lambda c: (c, 0, 0)),
            pl.BlockSpec((B, R), lambda c: (0, 0)),
            pl.BlockSpec((n_in, R), lambda c: (0, 0)),
            pl.BlockSpec((R, R), lambda c: (0, 0)),
            pl.BlockSpec((R, n_in), lambda c: (0, 0)),
        ],
        out_specs=[
            pl.BlockSpec((tt, B, R), lambda c: (c, 0, 0)),
            pl.BlockSpec((tt, B, n_in), lambda c: (c, 0, 0)),
        ],
        scratch_shapes=[
            pltpu.VMEM((B, R), jnp.float32),
            pltpu.VMEM((R + n_in, R), jnp.float32),
            pltpu.VMEM((R, 2 * n_in), jnp.bfloat16),
        ],
        compiler_params=pltpu.CompilerParams(
            dimension_semantics=("arbitrary",)),
    )(x_seq, h0, win_t, wr_t, wout_t)
    return out_seq, h_seq


def kernel(x_seq, h0, win_t, wr_t, wout_t):
    return _esn_forward(x_seq, h0, win_t, wr_t, wout_t)
